# baseline (speedup 1.0000x reference)
import functools

import jax
import jax.numpy as jnp
from jax import lax
from jax.experimental import pallas as pl
from jax.experimental.pallas import tpu as pltpu
from jax.experimental.pallas import tpu_sc as plsc

NUM_CORES = 2
NUM_SUBCORES = 16
NUM_WORKERS = NUM_CORES * NUM_SUBCORES
LANES = 16
BATCH = 16384
HIDDEN = 64
B_PER_W = BATCH // NUM_WORKERS
CHUNK = 128
NCHUNK = B_PER_W // CHUNK
KREG = HIDDEN // LANES
PSTRIDE = B_PER_W + 1


def _sc_body(head_hbm, rel_hbm, tail_hbm, node_hbm, relemb_hbm, out_hbm,
             hidx, ridx, tidx, h_rows, r_rows, t_rows, pt, out_v, sem):
    w = lax.axis_index("s") * NUM_CORES + lax.axis_index("c")
    idx_row = w * NCHUNK

    pltpu.sync_copy(head_hbm.at[pl.ds(idx_row, NCHUNK)], hidx)
    pltpu.sync_copy(rel_hbm.at[pl.ds(idx_row, NCHUNK)], ridx)
    pltpu.sync_copy(tail_hbm.at[pl.ds(idx_row, NCHUNK)], tidx)

    copies = []
    for k in range(NCHUNK):
        dst = pl.ds(k * CHUNK, CHUNK)
        copies.append(pltpu.async_copy(node_hbm.at[hidx.at[k]], h_rows.at[dst], sem))
        copies.append(pltpu.async_copy(relemb_hbm.at[ridx.at[k]], r_rows.at[dst], sem))
        copies.append(pltpu.async_copy(node_hbm.at[tidx.at[k]], t_rows.at[dst], sem))
    for c in copies:
        c.wait()

    iota = lax.iota(jnp.int32, LANES)

    @pl.loop(0, B_PER_W)
    def _row(i):
        acc = None
        for k in range(KREG):
            d = pl.ds(k * LANES, LANES)
            v = jnp.abs(h_rows[i, d] + r_rows[i, d] - t_rows[i, d])
            acc = v if acc is None else acc + v
        plsc.store_scatter(pt, [iota * PSTRIDE + i], acc)

    @pl.loop(0, B_PER_W // LANES)
    def _grp(g):
        base = g * LANES
        s = pt[pl.ds(base, LANES)]
        for l in range(1, LANES):
            s = s + pt[pl.ds(l * PSTRIDE + base, LANES)]
        out_v[pl.ds(base, LANES)] = -s

    pltpu.sync_copy(out_v, out_hbm.at[pl.ds(w * B_PER_W, B_PER_W)])


_mesh = plsc.VectorSubcoreMesh(
    core_axis_name="c", subcore_axis_name="s",
    num_cores=NUM_CORES, num_subcores=NUM_SUBCORES)

_sc_call = functools.partial(
    pl.kernel,
    out_type=jax.ShapeDtypeStruct((BATCH,), jnp.float32),
    mesh=_mesh,
    compiler_params=pltpu.CompilerParams(
        needs_layout_passes=False, use_tc_tiling_on_sc=False),
    scratch_types=[
        pltpu.VMEM((NCHUNK, CHUNK), jnp.int32),
        pltpu.VMEM((NCHUNK, CHUNK), jnp.int32),
        pltpu.VMEM((NCHUNK, CHUNK), jnp.int32),
        pltpu.VMEM((B_PER_W, HIDDEN), jnp.float32),
        pltpu.VMEM((B_PER_W, HIDDEN), jnp.float32),
        pltpu.VMEM((B_PER_W, HIDDEN), jnp.float32),
        pltpu.VMEM((LANES * PSTRIDE,), jnp.float32),
        pltpu.VMEM((B_PER_W,), jnp.float32),
        pltpu.SemaphoreType.DMA,
    ],
)(_sc_body)


@jax.jit
def kernel(head_index, rel_type, tail_index, node_emb, rel_emb):
    h = head_index.astype(jnp.int32).reshape(BATCH // CHUNK, CHUNK)
    r = rel_type.astype(jnp.int32).reshape(BATCH // CHUNK, CHUNK)
    t = tail_index.astype(jnp.int32).reshape(BATCH // CHUNK, CHUNK)
    return _sc_call(h, r, t, node_emb, rel_emb)
